# baseline (speedup 1.0000x reference)
import functools

import jax
import jax.numpy as jnp
from jax import lax
from jax.experimental import pallas as pl
from jax.experimental.pallas import tpu as pltpu
from jax.experimental.pallas import tpu_sc as plsc

N_EXP = 8
BLK = 256
FBLK = 768
ROWB = 1024
SC_NW = 32
SC_CH = 64


def _gating_body(x_ref, wg_ref, w0_ref, w1_ref, i0_ref, i1_ref,
                 r0_ref, r1_ref, c0_ref, ct_ref, carry0, carry1):
    r = pl.program_id(0)
    nr = pl.num_programs(0)

    @pl.when(r == 0)
    def _():
        carry0[...] = jnp.zeros_like(carry0)
        carry1[...] = jnp.zeros_like(carry1)

    logits = jnp.dot(x_ref[...], wg_ref[...], preferred_element_type=jnp.float32)
    e_dim = logits.shape[-1]
    lmax = jnp.max(logits, axis=-1, keepdims=True)
    ex = jnp.exp(logits - lmax)
    iota = lax.broadcasted_iota(jnp.int32, ex.shape, 1)
    m0 = jnp.max(ex, axis=-1, keepdims=True)
    i0 = jnp.min(jnp.where(ex == m0, iota, e_dim), axis=-1, keepdims=True)
    ex2 = jnp.where(iota == i0, -1.0, ex)
    m1 = jnp.max(ex2, axis=-1, keepdims=True)
    i1 = jnp.min(jnp.where(ex2 == m1, iota, e_dim), axis=-1, keepdims=True)
    s = m0 + m1
    w0_ref[...] = m0 / s
    w1_ref[...] = m1 / s
    i0_ref[...] = i0
    i1_ref[...] = i1

    rows = logits.shape[0]
    ri = lax.broadcasted_iota(jnp.int32, (rows, rows), 0)
    ci = lax.broadcasted_iota(jnp.int32, (rows, rows), 1)
    tri = jnp.where(ci < ri, 1.0, 0.0)
    oh0 = jnp.where(iota == i0, 1.0, 0.0)
    oh1 = jnp.where(iota == i1, 1.0, 0.0)
    pre0 = jnp.dot(tri, oh0, preferred_element_type=jnp.float32) + carry0[...]
    pre1 = jnp.dot(tri, oh1, preferred_element_type=jnp.float32) + carry1[...]
    r0_ref[...] = jnp.sum(oh0 * pre0, axis=-1, keepdims=True).astype(jnp.int32)
    r1_ref[...] = jnp.sum(oh1 * pre1, axis=-1, keepdims=True).astype(jnp.int32)
    carry0[...] += jnp.sum(oh0, axis=0, keepdims=True)
    carry1[...] += jnp.sum(oh1, axis=0, keepdims=True)

    @pl.when(r == nr - 1)
    def _():
        c0_ref[...] = carry0[...].astype(jnp.int32)
        ct_ref[...] = (carry0[...] + carry1[...]).astype(jnp.int32)


def _gating(flat, Wg):
    S, D = flat.shape
    E = Wg.shape[1]
    o_spec = pl.BlockSpec((ROWB, 1), lambda r: (r, 0))
    c_spec = pl.BlockSpec((1, E), lambda r: (0, 0))
    return pl.pallas_call(
        _gating_body,
        grid=(S // ROWB,),
        in_specs=[
            pl.BlockSpec((ROWB, D), lambda r: (r, 0)),
            pl.BlockSpec((D, E), lambda r: (0, 0)),
        ],
        out_specs=[o_spec, o_spec, o_spec, o_spec, o_spec, o_spec,
                   c_spec, c_spec],
        out_shape=[
            jax.ShapeDtypeStruct((S, 1), jnp.float32),
            jax.ShapeDtypeStruct((S, 1), jnp.float32),
            jax.ShapeDtypeStruct((S, 1), jnp.int32),
            jax.ShapeDtypeStruct((S, 1), jnp.int32),
            jax.ShapeDtypeStruct((S, 1), jnp.int32),
            jax.ShapeDtypeStruct((S, 1), jnp.int32),
            jax.ShapeDtypeStruct((1, E), jnp.int32),
            jax.ShapeDtypeStruct((1, E), jnp.int32),
        ],
        scratch_shapes=[
            pltpu.VMEM((1, E), jnp.float32),
            pltpu.VMEM((1, E), jnp.float32),
        ],
    )(flat, Wg)


def _route(i0, i1, r0, r1, w0, w1, c0, ct, nb):
    S = i0.shape[0]
    padded = ((ct + BLK - 1) // BLK) * BLK
    pcum = jnp.cumsum(padded)
    pstart = pcum - padded
    pos0 = pstart[i0] + r0
    pos1 = pstart[i1] + c0[i1] + r1
    npad = nb * BLK
    ar = jnp.arange(S, dtype=jnp.int32)
    allpos = jnp.concatenate([pos0, pos1])
    gidx = (jnp.arange(npad, dtype=jnp.int32) % S).at[allpos].set(
        jnp.concatenate([ar, ar]), unique_indices=True)
    gw = jnp.zeros((npad, 1), jnp.float32).at[allpos].set(
        jnp.concatenate([w0, w1])[:, None], unique_indices=True)
    bcum = pcum // BLK
    block_expert = jnp.searchsorted(
        bcum, jnp.arange(nb, dtype=jnp.int32), side="right"
    ).astype(jnp.int32)
    block_expert = jnp.minimum(block_expert, N_EXP - 1)
    nb_used = bcum[-1].reshape(1).astype(jnp.int32)
    return gidx, gw, allpos, block_expert, nb_used


def _sc_gather(table, idx):
    V, D = table.shape
    B = idx.shape[0]
    b_per_w = B // SC_NW
    ch = max(c for c in (128, 96, 80, 64, 40, 32, 16, 8)
             if b_per_w % c == 0 and c * D * 4 <= 450_000) if b_per_w > 128 \
        else b_per_w
    n_ch = b_per_w // ch
    mesh = plsc.VectorSubcoreMesh(core_axis_name="c", subcore_axis_name="s")

    @functools.partial(
        pl.kernel,
        mesh=mesh,
        out_type=jax.ShapeDtypeStruct((B, D), jnp.float32),
        scratch_types=[
            pltpu.VMEM((ch,), jnp.int32),
            pltpu.VMEM((ch, D), jnp.float32),
            pltpu.SemaphoreType.DMA,
        ],
    )
    def k(table_hbm, idx_hbm, out_hbm, idx_v, rows_v, sem):
        wid = lax.axis_index("s") * 2 + lax.axis_index("c")
        base = wid * b_per_w
        for c in range(n_ch):
            off = base + c * ch
            pltpu.sync_copy(idx_hbm.at[pl.ds(off, ch)], idx_v)
            pltpu.async_copy(table_hbm.at[idx_v], rows_v, sem).wait()
            pltpu.sync_copy(rows_v, out_hbm.at[pl.ds(off, ch)])

    return k(table, idx)


def _ffn_body(be_ref, nbu_ref, xg_ref, w1_ref, b1_ref, w2_ref, b2_ref,
              gw_ref, yw_ref):
    b = pl.program_id(0)

    @pl.when(b < nbu_ref[0])
    def _():
        h = jnp.dot(xg_ref[...], w1_ref[0], preferred_element_type=jnp.float32)
        h = jax.nn.gelu(h + b1_ref[0])
        yw_ref[...] = (
            jnp.dot(h, w2_ref[0], preferred_element_type=jnp.float32)
            + b2_ref[0]
        ) * gw_ref[...]


def _ffn(xg, W1, b1, W2, b2, gw, block_expert, nb_used):
    npad, D = xg.shape
    F = W1.shape[2]
    nb = npad // BLK

    def _bb(b, nbu):
        return jnp.minimum(b, nbu[0] - 1)

    grid_spec = pltpu.PrefetchScalarGridSpec(
        num_scalar_prefetch=2,
        grid=(nb,),
        in_specs=[
            pl.BlockSpec((BLK, D), lambda b, be, nbu: (_bb(b, nbu), 0)),
            pl.BlockSpec((1, D, F), lambda b, be, nbu: (be[_bb(b, nbu)], 0, 0)),
            pl.BlockSpec((1, 1, F), lambda b, be, nbu: (be[_bb(b, nbu)], 0, 0)),
            pl.BlockSpec((1, F, D), lambda b, be, nbu: (be[_bb(b, nbu)], 0, 0)),
            pl.BlockSpec((1, 1, D), lambda b, be, nbu: (be[_bb(b, nbu)], 0, 0)),
            pl.BlockSpec((BLK, 1), lambda b, be, nbu: (_bb(b, nbu), 0)),
        ],
        out_specs=pl.BlockSpec((BLK, D), lambda b, be, nbu: (_bb(b, nbu), 0)),
    )
    return pl.pallas_call(
        _ffn_body,
        grid_spec=grid_spec,
        out_shape=jax.ShapeDtypeStruct((npad, D), jnp.float32),
    )(block_expert, nb_used, xg, W1, b1.reshape(N_EXP, 1, F),
      W2, b2.reshape(N_EXP, 1, D), gw)


def _sc_combine(yw, allpos):
    _, D = yw.shape
    S = allpos.shape[0] // 2
    t_per_w = S // SC_NW
    mesh = plsc.VectorSubcoreMesh(core_axis_name="c", subcore_axis_name="s")
    nseg = D // 16

    @functools.partial(
        pl.kernel,
        mesh=mesh,
        out_type=jax.ShapeDtypeStruct((S, D), jnp.float32),
        scratch_types=[
            pltpu.VMEM((t_per_w,), jnp.int32),
            pltpu.VMEM((t_per_w,), jnp.int32),
            pltpu.VMEM((t_per_w, D), jnp.float32),
            pltpu.VMEM((t_per_w, D), jnp.float32),
            pltpu.SemaphoreType.DMA,
            pltpu.SemaphoreType.DMA,
        ],
    )
    def k(yw_hbm, pos_hbm, out_hbm, i0v, i1v, r0v, r1v, s0, s1):
        wid = lax.axis_index("s") * 2 + lax.axis_index("c")
        base = wid * t_per_w
        pltpu.sync_copy(pos_hbm.at[pl.ds(base, t_per_w)], i0v)
        pltpu.sync_copy(pos_hbm.at[pl.ds(S + base, t_per_w)], i1v)
        c0 = pltpu.async_copy(yw_hbm.at[i0v], r0v, s0)
        c1 = pltpu.async_copy(yw_hbm.at[i1v], r1v, s1)
        c0.wait()
        c1.wait()

        def row(i, carry):
            for j in range(nseg):
                sl = pl.ds(j * 16, 16)
                r0v[i, sl] += r1v[i, sl]
            return carry

        lax.fori_loop(0, t_per_w, row, 0)
        pltpu.sync_copy(r0v, out_hbm.at[pl.ds(base, t_per_w)])

    return k(yw, allpos)


def _comb_body(a_ref, b_ref, w0_ref, w1_ref, o_ref):
    o_ref[...] = w0_ref[...] * a_ref[...] + w1_ref[...] * b_ref[...]


def _combine(yg, w0, w1):
    S2, D = yg.shape
    S = S2 // 2
    nr = S // ROWB
    spec = pl.BlockSpec((ROWB, D), lambda r: (r, 0))
    w_spec = pl.BlockSpec((ROWB, 1), lambda r: (r, 0))
    return pl.pallas_call(
        _comb_body,
        grid=(nr,),
        in_specs=[
            pl.BlockSpec((ROWB, D), lambda r: (r, 0)),
            pl.BlockSpec((ROWB, D), lambda r: (r + nr, 0)),
            w_spec, w_spec,
        ],
        out_specs=spec,
        out_shape=jax.ShapeDtypeStruct((S, D), jnp.float32),
    )(yg, yg, w0, w1)


def kernel(x, Wg, W1, b1, W2, b2):
    B, S, D = x.shape
    flat = x.reshape(B * S, D)
    w0, w1, i0, i1, r0, r1, c0, ct = _gating(flat, Wg)
    nb = (2 * B * S) // BLK + N_EXP
    gidx, gw, allpos, block_expert, nb_used = _route(
        i0[:, 0], i1[:, 0], r0[:, 0], r1[:, 0], w0[:, 0], w1[:, 0],
        c0[0], ct[0], nb
    )
    xg = _sc_gather(flat, gidx)
    yw = _ffn(xg, W1, b1, W2, b2, gw, block_expert, nb_used)
    out = _sc_combine(yw, allpos)
    return out.reshape(B, S, D)


# speedup vs baseline: 1.0067x; 1.0067x over previous
import functools

import jax
import jax.numpy as jnp
from jax import lax
from jax.experimental import pallas as pl
from jax.experimental.pallas import tpu as pltpu
from jax.experimental.pallas import tpu_sc as plsc

N_EXP = 8
BLK = 256
FBLK = 768
ROWB = 1024
SC_NW = 32
SC_CH = 64


def _gating_body(x_ref, wg_ref, w0_ref, w1_ref, i0_ref, i1_ref,
                 r0_ref, r1_ref, c0_ref, ct_ref, carry0, carry1):
    r = pl.program_id(0)
    nr = pl.num_programs(0)

    @pl.when(r == 0)
    def _():
        carry0[...] = jnp.zeros_like(carry0)
        carry1[...] = jnp.zeros_like(carry1)

    logits = jnp.dot(x_ref[...], wg_ref[...], preferred_element_type=jnp.float32)
    e_dim = logits.shape[-1]
    lmax = jnp.max(logits, axis=-1, keepdims=True)
    ex = jnp.exp(logits - lmax)
    iota = lax.broadcasted_iota(jnp.int32, ex.shape, 1)
    m0 = jnp.max(ex, axis=-1, keepdims=True)
    i0 = jnp.min(jnp.where(ex == m0, iota, e_dim), axis=-1, keepdims=True)
    ex2 = jnp.where(iota == i0, -1.0, ex)
    m1 = jnp.max(ex2, axis=-1, keepdims=True)
    i1 = jnp.min(jnp.where(ex2 == m1, iota, e_dim), axis=-1, keepdims=True)
    s = m0 + m1
    w0_ref[...] = m0 / s
    w1_ref[...] = m1 / s
    i0_ref[...] = i0
    i1_ref[...] = i1

    rows = logits.shape[0]
    ri = lax.broadcasted_iota(jnp.int32, (rows, rows), 0)
    ci = lax.broadcasted_iota(jnp.int32, (rows, rows), 1)
    tri = jnp.where(ci < ri, 1.0, 0.0)
    oh0 = jnp.where(iota == i0, 1.0, 0.0)
    oh1 = jnp.where(iota == i1, 1.0, 0.0)
    pre0 = jnp.dot(tri, oh0, preferred_element_type=jnp.float32) + carry0[...]
    pre1 = jnp.dot(tri, oh1, preferred_element_type=jnp.float32) + carry1[...]
    r0_ref[...] = jnp.sum(oh0 * pre0, axis=-1, keepdims=True).astype(jnp.int32)
    r1_ref[...] = jnp.sum(oh1 * pre1, axis=-1, keepdims=True).astype(jnp.int32)
    carry0[...] += jnp.sum(oh0, axis=0, keepdims=True)
    carry1[...] += jnp.sum(oh1, axis=0, keepdims=True)

    @pl.when(r == nr - 1)
    def _():
        c0_ref[...] = carry0[...].astype(jnp.int32)
        ct_ref[...] = (carry0[...] + carry1[...]).astype(jnp.int32)


def _gating(flat, Wg):
    S, D = flat.shape
    E = Wg.shape[1]
    o_spec = pl.BlockSpec((ROWB, 1), lambda r: (r, 0))
    c_spec = pl.BlockSpec((1, E), lambda r: (0, 0))
    return pl.pallas_call(
        _gating_body,
        grid=(S // ROWB,),
        in_specs=[
            pl.BlockSpec((ROWB, D), lambda r: (r, 0)),
            pl.BlockSpec((D, E), lambda r: (0, 0)),
        ],
        out_specs=[o_spec, o_spec, o_spec, o_spec, o_spec, o_spec,
                   c_spec, c_spec],
        out_shape=[
            jax.ShapeDtypeStruct((S, 1), jnp.float32),
            jax.ShapeDtypeStruct((S, 1), jnp.float32),
            jax.ShapeDtypeStruct((S, 1), jnp.int32),
            jax.ShapeDtypeStruct((S, 1), jnp.int32),
            jax.ShapeDtypeStruct((S, 1), jnp.int32),
            jax.ShapeDtypeStruct((S, 1), jnp.int32),
            jax.ShapeDtypeStruct((1, E), jnp.int32),
            jax.ShapeDtypeStruct((1, E), jnp.int32),
        ],
        scratch_shapes=[
            pltpu.VMEM((1, E), jnp.float32),
            pltpu.VMEM((1, E), jnp.float32),
        ],
    )(flat, Wg)


def _route(i0, i1, r0, r1, c0, ct, nb):
    S = i0.shape[0]
    padded = ((ct + BLK - 1) // BLK) * BLK
    pcum = jnp.cumsum(padded)
    pstart = pcum - padded
    pos0 = pstart[i0] + r0
    pos1 = pstart[i1] + c0[i1] + r1
    npad = nb * BLK
    ar = jnp.arange(S, dtype=jnp.int32)
    gidx = (jnp.arange(npad, dtype=jnp.int32) % S).at[
        jnp.concatenate([pos0, pos1])
    ].set(jnp.concatenate([ar, ar]), unique_indices=True)
    bcum = pcum // BLK
    block_expert = jnp.searchsorted(
        bcum, jnp.arange(nb, dtype=jnp.int32), side="right"
    ).astype(jnp.int32)
    block_expert = jnp.minimum(block_expert, N_EXP - 1)
    nb_used = bcum[-1].reshape(1).astype(jnp.int32)
    return gidx, pos0, pos1, block_expert, nb_used


def _sc_gather(table, idx):
    V, D = table.shape
    B = idx.shape[0]
    b_per_w = B // SC_NW
    ch = max(c for c in (128, 96, 80, 64, 40, 32, 16, 8)
             if b_per_w % c == 0 and c * D * 4 <= 450_000) if b_per_w > 128 \
        else b_per_w
    n_ch = b_per_w // ch
    mesh = plsc.VectorSubcoreMesh(core_axis_name="c", subcore_axis_name="s")

    @functools.partial(
        pl.kernel,
        mesh=mesh,
        out_type=jax.ShapeDtypeStruct((B, D), jnp.float32),
        scratch_types=[
            pltpu.VMEM((ch,), jnp.int32),
            pltpu.VMEM((ch, D), jnp.float32),
            pltpu.SemaphoreType.DMA,
        ],
    )
    def k(table_hbm, idx_hbm, out_hbm, idx_v, rows_v, sem):
        wid = lax.axis_index("s") * 2 + lax.axis_index("c")
        base = wid * b_per_w
        for c in range(n_ch):
            off = base + c * ch
            pltpu.sync_copy(idx_hbm.at[pl.ds(off, ch)], idx_v)
            pltpu.async_copy(table_hbm.at[idx_v], rows_v, sem).wait()
            pltpu.sync_copy(rows_v, out_hbm.at[pl.ds(off, ch)])

    return k(table, idx)


def _ffn_body(be_ref, nbu_ref, xg_ref, w1_ref, b1_ref, w2_ref, b2_ref,
              yw_ref):
    b = pl.program_id(0)

    @pl.when(b < nbu_ref[0])
    def _():
        h = jnp.dot(xg_ref[...], w1_ref[0], preferred_element_type=jnp.float32)
        h = jax.nn.gelu(h + b1_ref[0])
        yw_ref[...] = (
            jnp.dot(h, w2_ref[0], preferred_element_type=jnp.float32)
            + b2_ref[0]
        )


def _ffn(xg, W1, b1, W2, b2, block_expert, nb_used):
    npad, D = xg.shape
    F = W1.shape[2]
    nb = npad // BLK

    def _bb(b, nbu):
        return jnp.minimum(b, nbu[0] - 1)

    grid_spec = pltpu.PrefetchScalarGridSpec(
        num_scalar_prefetch=2,
        grid=(nb,),
        in_specs=[
            pl.BlockSpec((BLK, D), lambda b, be, nbu: (_bb(b, nbu), 0)),
            pl.BlockSpec((1, D, F), lambda b, be, nbu: (be[_bb(b, nbu)], 0, 0)),
            pl.BlockSpec((1, 1, F), lambda b, be, nbu: (be[_bb(b, nbu)], 0, 0)),
            pl.BlockSpec((1, F, D), lambda b, be, nbu: (be[_bb(b, nbu)], 0, 0)),
            pl.BlockSpec((1, 1, D), lambda b, be, nbu: (be[_bb(b, nbu)], 0, 0)),
        ],
        out_specs=pl.BlockSpec((BLK, D), lambda b, be, nbu: (_bb(b, nbu), 0)),
    )
    return pl.pallas_call(
        _ffn_body,
        grid_spec=grid_spec,
        out_shape=jax.ShapeDtypeStruct((npad, D), jnp.float32),
    )(block_expert, nb_used, xg, W1, b1.reshape(N_EXP, 1, F),
      W2, b2.reshape(N_EXP, 1, D))


def _comb_body(a_ref, b_ref, w0_ref, w1_ref, o_ref):
    o_ref[...] = w0_ref[...] * a_ref[...] + w1_ref[...] * b_ref[...]


def _combine(yg, w0, w1):
    S2, D = yg.shape
    S = S2 // 2
    nr = S // ROWB
    spec = pl.BlockSpec((ROWB, D), lambda r: (r, 0))
    w_spec = pl.BlockSpec((ROWB, 1), lambda r: (r, 0))
    return pl.pallas_call(
        _comb_body,
        grid=(nr,),
        in_specs=[
            pl.BlockSpec((ROWB, D), lambda r: (r, 0)),
            pl.BlockSpec((ROWB, D), lambda r: (r + nr, 0)),
            w_spec, w_spec,
        ],
        out_specs=spec,
        out_shape=jax.ShapeDtypeStruct((S, D), jnp.float32),
    )(yg, yg, w0, w1)


def kernel(x, Wg, W1, b1, W2, b2):
    B, S, D = x.shape
    flat = x.reshape(B * S, D)
    w0, w1, i0, i1, r0, r1, c0, ct = _gating(flat, Wg)
    nb = (2 * B * S) // BLK + N_EXP
    gidx, pos0, pos1, block_expert, nb_used = _route(
        i0[:, 0], i1[:, 0], r0[:, 0], r1[:, 0], c0[0], ct[0], nb
    )
    xg = _sc_gather(flat, gidx)
    yw = _ffn(xg, W1, b1, W2, b2, block_expert, nb_used)
    yg = _sc_gather(yw, jnp.concatenate([pos0, pos1]))
    out = _combine(yg, w0, w1)
    return out.reshape(B, S, D)
